# baseline (speedup 1.0000x reference)
import functools

import jax
import jax.numpy as jnp
from jax import lax
from jax.experimental import pallas as pl
from jax.experimental.pallas import tpu as pltpu
from jax.experimental.pallas import tpu_sc as plsc

N = 10000
NP = 10240
D = 256
HID = 256
C = 64
E = 160000
EP = 163840
NC, NS = 2, 16
CH = 128
RPS = NP // NS

_MESH = plsc.VectorSubcoreMesh(
    core_axis_name="c", subcore_axis_name="s", num_cores=NC, num_subcores=NS)


@functools.partial(
    pl.kernel,
    out_type=jax.ShapeDtypeStruct((NC * NP, 128), jnp.float32),
    mesh=_MESH,
    scratch_types=[
        pltpu.VMEM((CH,), jnp.int32),
        pltpu.VMEM((CH,), jnp.int32),
        pltpu.VMEM((CH, 128), jnp.float32),
        pltpu.VMEM_SHARED((NP, 128), jnp.float32),
        pltpu.SemaphoreType.DMA,
        pltpu.SemaphoreType.DMA,
    ],
)
def _sc_degree(row_hbm, ones_hbm, zero_hbm, out_hbm,
               row_v0, row_v1, ones_v, acc, sem0, sem1):
    c = lax.axis_index("c")
    s = lax.axis_index("s")
    nch = EP // (NC * NS) // CH
    pltpu.sync_copy(zero_hbm, acc.at[pl.ds(s * RPS, RPS)])
    pltpu.sync_copy(ones_hbm, ones_v)
    plsc.subcore_barrier()
    base = c * (EP // NC) + s * (EP // (NC * NS))
    rows = (row_v0, row_v1)
    sems = (sem0, sem1)

    @pl.loop(0, nch, step=2)
    def _(j):
        for b in range(2):
            jj = j + b

            @pl.when(jj >= 2)
            def _():
                pltpu.make_async_copy(ones_v, acc.at[rows[b]], sems[b]).wait()

            pltpu.sync_copy(row_hbm.at[pl.ds(base + jj * CH, CH)], rows[b])
            pltpu.async_copy(ones_v, acc.at[rows[b]], sems[b], add=True)

    pltpu.make_async_copy(ones_v, acc.at[row_v0], sem0).wait()
    pltpu.make_async_copy(ones_v, acc.at[row_v1], sem1).wait()
    plsc.subcore_barrier()
    pltpu.sync_copy(acc.at[pl.ds(s * RPS, RPS)],
                    out_hbm.at[pl.ds(c * NP + s * RPS, RPS)])


_NB = 4


def _agg_body(nch):
    def body(g_hbm, pack_hbm, zero_hbm, out_hbm,
             idxs, buf, acc, isems, gsems, ssems, kbase, s):
        pltpu.sync_copy(zero_hbm, acc.at[pl.ds(s * RPS, RPS)])
        plsc.subcore_barrier()

        def idx_load(k, slot):
            pltpu.async_copy(pack_hbm.at[kbase + k], idxs[slot], isems[slot])

        def gather(k, bslot, qslot):
            pltpu.async_copy(g_hbm.at[idxs[qslot].at[0]], buf.at[bslot],
                             gsems[bslot])

        idx_load(0, 0)
        idx_load(1, 1)
        pltpu.make_async_copy(pack_hbm.at[kbase], idxs[0], isems[0]).wait()
        gather(0, 0, 0)

        @pl.loop(0, nch, step=_NB)
        def _(j):
            for b in range(_NB):
                jj = j + b
                bb = b % 2
                b1 = (b + 1) % 2
                q1 = (b + 1) % _NB
                q2 = (b + 2) % _NB

                @pl.when(jj + 2 < nch)
                def _():
                    idx_load(jj + 2, q2)

                @pl.when(jj + 1 < nch)
                def _():
                    @pl.when(jj >= 1)
                    def _():
                        pltpu.make_async_copy(
                            buf.at[b1], acc.at[idxs[q1].at[1]],
                            ssems[b1]).wait()
                    pltpu.make_async_copy(
                        pack_hbm.at[kbase], idxs[q1], isems[q1]).wait()
                    gather(jj + 1, b1, q1)

                pltpu.make_async_copy(
                    g_hbm.at[idxs[b].at[0]], buf.at[bb], gsems[bb]).wait()
                pltpu.async_copy(buf.at[bb], acc.at[idxs[b].at[1]], ssems[bb],
                                 add=True)

        for k in (nch - 2, nch - 1):
            pltpu.make_async_copy(
                buf.at[k % 2], acc.at[idxs[k % _NB].at[1]],
                ssems[k % 2]).wait()
        plsc.subcore_barrier()

    return body


_AGG_SCRATCH = (
    [pltpu.VMEM((2, CH), jnp.int32) for _ in range(_NB)]
    + [pltpu.VMEM((2, CH, 128), jnp.float32),
       pltpu.VMEM_SHARED((NP, 128), jnp.float32)]
    + [pltpu.SemaphoreType.DMA] * _NB
    + [pltpu.SemaphoreType.DMA] * 2
    + [pltpu.SemaphoreType.DMA] * 2
)

_KCH = EP // CH


@functools.partial(
    pl.kernel,
    out_type=jax.ShapeDtypeStruct((NC * NP, 128), jnp.float32),
    mesh=_MESH,
    scratch_types=_AGG_SCRATCH,
)
def _sc_agg2(g_hbm, pack_hbm, zero_hbm, out_hbm, *sc):
    idxs = sc[:_NB]
    buf, acc = sc[_NB], sc[_NB + 1]
    isems = sc[_NB + 2:2 * _NB + 2]
    gsems = sc[2 * _NB + 2:2 * _NB + 4]
    ssems = sc[2 * _NB + 4:2 * _NB + 6]
    c = lax.axis_index("c")
    s = lax.axis_index("s")
    kbase = c * (_KCH // NC) + s * (_KCH // (NC * NS))
    _agg_body(_KCH // (NC * NS))(g_hbm, pack_hbm, zero_hbm, out_hbm,
                                 idxs, buf, acc, isems, gsems, ssems, kbase, s)
    pltpu.sync_copy(acc.at[pl.ds(s * RPS, RPS)],
                    out_hbm.at[pl.ds(c * NP + s * RPS, RPS)])


_BN = 256
_GRID = (NP // _BN,)


def _p_from_deg(dref):
    deg = dref[0, :, 0] + dref[1, :, 0] + 1.0 + 1e-8
    return lax.rsqrt(deg)


def _tc1_body(x_ref, w_ref, d_ref, h_ref, g_ref):
    p = _p_from_deg(d_ref)
    h = jnp.dot(x_ref[...], w_ref[...], preferred_element_type=jnp.float32)
    h_ref[...] = h
    g = h * p[:, None]
    g_ref[0] = g[:, :128]
    g_ref[1] = g[:, 128:]


_tc1 = pl.pallas_call(
    _tc1_body,
    grid=_GRID,
    in_specs=[
        pl.BlockSpec((_BN, D), lambda i: (i, 0)),
        pl.BlockSpec((D, HID), lambda i: (0, 0)),
        pl.BlockSpec((2, _BN, 16), lambda i: (0, i, 0)),
    ],
    out_specs=[
        pl.BlockSpec((_BN, HID), lambda i: (i, 0)),
        pl.BlockSpec((2, _BN, 128), lambda i: (0, i, 0)),
    ],
    out_shape=[
        jax.ShapeDtypeStruct((NP, HID), jnp.float32),
        jax.ShapeDtypeStruct((2, NP, 128), jnp.float32),
    ],
)


def _tc2_body(aa_ref, ab_ref, h1_ref, d_ref, b_ref, w_ref, h2_ref, g2_ref):
    p = _p_from_deg(d_ref)
    agg = jnp.concatenate([aa_ref[0] + aa_ref[1], ab_ref[0] + ab_ref[1]],
                          axis=1)
    z = p[:, None] * agg + (p * p)[:, None] * h1_ref[...] + b_ref[...]
    z = jnp.maximum(z, 0.0)
    h2 = jnp.dot(z, w_ref[...], preferred_element_type=jnp.float32)
    h2_ref[...] = h2
    g2_ref[:, :C] = h2 * p[:, None]
    g2_ref[:, C:] = jnp.zeros((_BN, 128 - C), jnp.float32)


_tc2 = pl.pallas_call(
    _tc2_body,
    grid=_GRID,
    in_specs=[
        pl.BlockSpec((2, _BN, 128), lambda i: (0, i, 0)),
        pl.BlockSpec((2, _BN, 128), lambda i: (0, i, 0)),
        pl.BlockSpec((_BN, HID), lambda i: (i, 0)),
        pl.BlockSpec((2, _BN, 16), lambda i: (0, i, 0)),
        pl.BlockSpec((1, HID), lambda i: (0, 0)),
        pl.BlockSpec((HID, C), lambda i: (0, 0)),
    ],
    out_specs=[
        pl.BlockSpec((_BN, C), lambda i: (i, 0)),
        pl.BlockSpec((_BN, 128), lambda i: (i, 0)),
    ],
    out_shape=[
        jax.ShapeDtypeStruct((NP, C), jnp.float32),
        jax.ShapeDtypeStruct((NP, 128), jnp.float32),
    ],
)


def _tc3_body(a_ref, h2_ref, d_ref, b_ref, o_ref):
    p = _p_from_deg(d_ref)
    z = (p[:, None] * (a_ref[0, :, :C] + a_ref[1, :, :C])
         + (p * p)[:, None] * h2_ref[...] + b_ref[...])
    m = jnp.max(z, axis=-1, keepdims=True)
    e = jnp.exp(z - m)
    lse = jnp.log(jnp.sum(e, axis=-1, keepdims=True))
    o_ref[...] = z - m - lse


_tc3 = pl.pallas_call(
    _tc3_body,
    grid=_GRID,
    in_specs=[
        pl.BlockSpec((2, _BN, 128), lambda i: (0, i, 0)),
        pl.BlockSpec((_BN, C), lambda i: (i, 0)),
        pl.BlockSpec((2, _BN, 16), lambda i: (0, i, 0)),
        pl.BlockSpec((1, C), lambda i: (0, 0)),
    ],
    out_specs=pl.BlockSpec((_BN, C), lambda i: (i, 0)),
    out_shape=jax.ShapeDtypeStruct((NP, C), jnp.float32),
)


def kernel(x, edge_index, W1, b1, W2, b2):
    row = edge_index[0].astype(jnp.int32)
    col = edge_index[1].astype(jnp.int32)
    pad = jnp.full((EP - E,), NP - 1, jnp.int32)
    rowp = jnp.concatenate([row, pad])
    colp = jnp.concatenate([col, pad])
    kc = colp.reshape(_KCH, CH)
    kr = rowp.reshape(_KCH, CH)
    pack2 = jnp.stack([kc, kr], axis=1)

    xp = jnp.pad(x, ((0, NP - N), (0, 0)))
    ones128 = jnp.ones((CH, 128), jnp.float32)
    z1 = jnp.zeros((RPS, 128), jnp.float32)

    degp = _sc_degree(rowp, ones128, z1).reshape(2, NP, 128)[:, :, :16]
    h1, g1 = _tc1(xp, W1, degp)
    acc1a = _sc_agg2(g1[0], pack2, z1).reshape(2, NP, 128)
    acc1b = _sc_agg2(g1[1], pack2, z1).reshape(2, NP, 128)
    h2, g2 = _tc2(acc1a, acc1b, h1, degp, b1.reshape(1, HID), W2)
    acc2 = _sc_agg2(g2, pack2, z1).reshape(2, NP, 128)
    out = _tc3(acc2, h2, degp, b2.reshape(1, C))
    return out[:N]


# speedup vs baseline: 1.1577x; 1.1577x over previous
import functools

import jax
import jax.numpy as jnp
from jax import lax
from jax.experimental import pallas as pl
from jax.experimental.pallas import tpu as pltpu
from jax.experimental.pallas import tpu_sc as plsc

N = 10000
NP = 10240
D = 256
HID = 256
C = 64
E = 160000
EP = 163840
NC, NS = 2, 16
CH = 128
RPS = NP // NS

_MESH = plsc.VectorSubcoreMesh(
    core_axis_name="c", subcore_axis_name="s", num_cores=NC, num_subcores=NS)


@functools.partial(
    pl.kernel,
    out_type=jax.ShapeDtypeStruct((NC * NP, 128), jnp.float32),
    mesh=_MESH,
    scratch_types=[
        pltpu.VMEM((CH,), jnp.int32),
        pltpu.VMEM((CH,), jnp.int32),
        pltpu.VMEM((CH, 128), jnp.float32),
        pltpu.VMEM_SHARED((NP, 128), jnp.float32),
        pltpu.SemaphoreType.DMA,
        pltpu.SemaphoreType.DMA,
    ],
)
def _sc_degree(row_hbm, ones_hbm, zero_hbm, out_hbm,
               row_v0, row_v1, ones_v, acc, sem0, sem1):
    c = lax.axis_index("c")
    s = lax.axis_index("s")
    nch = EP // (NC * NS) // CH
    pltpu.sync_copy(zero_hbm, acc.at[pl.ds(s * RPS, RPS)])
    pltpu.sync_copy(ones_hbm, ones_v)
    plsc.subcore_barrier()
    base = c * (EP // NC) + s * (EP // (NC * NS))
    rows = (row_v0, row_v1)
    sems = (sem0, sem1)

    @pl.loop(0, nch, step=2)
    def _(j):
        for b in range(2):
            jj = j + b

            @pl.when(jj >= 2)
            def _():
                pltpu.make_async_copy(ones_v, acc.at[rows[b]], sems[b]).wait()

            pltpu.sync_copy(row_hbm.at[pl.ds(base + jj * CH, CH)], rows[b])
            pltpu.async_copy(ones_v, acc.at[rows[b]], sems[b], add=True)

    pltpu.make_async_copy(ones_v, acc.at[row_v0], sem0).wait()
    pltpu.make_async_copy(ones_v, acc.at[row_v1], sem1).wait()
    plsc.subcore_barrier()
    pltpu.sync_copy(acc.at[pl.ds(s * RPS, RPS)],
                    out_hbm.at[pl.ds(c * NP + s * RPS, RPS)])


_NB = 4


def _agg_body(nch):
    def body(g_hbm, pack_hbm, zero_hbm, out_hbm,
             idxs, buf, acc, isems, gsems, ssems, kbase, s):
        pltpu.sync_copy(zero_hbm, acc.at[pl.ds(s * RPS, RPS)])
        plsc.subcore_barrier()

        def idx_load(k, slot):
            pltpu.async_copy(pack_hbm.at[kbase + k], idxs[slot], isems[slot])

        def gather(k, bslot, qslot):
            for h in range(2):
                pltpu.async_copy(
                    g_hbm.at[idxs[qslot].at[0, pl.ds(h * (CH // 2), CH // 2)]],
                    buf.at[bslot, pl.ds(h * (CH // 2), CH // 2)],
                    gsems[bslot])

        idx_load(0, 0)
        idx_load(1, 1)
        pltpu.make_async_copy(pack_hbm.at[kbase], idxs[0], isems[0]).wait()
        gather(0, 0, 0)

        @pl.loop(0, nch, step=_NB)
        def _(j):
            for b in range(_NB):
                jj = j + b
                bb = b % 2
                b1 = (b + 1) % 2
                q1 = (b + 1) % _NB
                q2 = (b + 2) % _NB

                @pl.when(jj + 2 < nch)
                def _():
                    idx_load(jj + 2, q2)

                @pl.when(jj + 1 < nch)
                def _():
                    @pl.when(jj >= 1)
                    def _():
                        pltpu.make_async_copy(
                            buf.at[b1], acc.at[idxs[q1].at[1]],
                            ssems[b1]).wait()
                    pltpu.make_async_copy(
                        pack_hbm.at[kbase], idxs[q1], isems[q1]).wait()
                    gather(jj + 1, b1, q1)

                pltpu.make_async_copy(
                    g_hbm.at[idxs[b].at[0]], buf.at[bb], gsems[bb]).wait()
                pltpu.async_copy(buf.at[bb], acc.at[idxs[b].at[1]], ssems[bb],
                                 add=True)

        for k in (nch - 2, nch - 1):
            pltpu.make_async_copy(
                buf.at[k % 2], acc.at[idxs[k % _NB].at[1]],
                ssems[k % 2]).wait()
        plsc.subcore_barrier()

    return body


_AGG_SCRATCH = (
    [pltpu.VMEM((2, CH), jnp.int32) for _ in range(_NB)]
    + [pltpu.VMEM((2, CH, 128), jnp.float32),
       pltpu.VMEM_SHARED((NP, 128), jnp.float32)]
    + [pltpu.SemaphoreType.DMA] * _NB
    + [pltpu.SemaphoreType.DMA] * 2
    + [pltpu.SemaphoreType.DMA] * 2
)

_KCH = EP // CH


@functools.partial(
    pl.kernel,
    out_type=jax.ShapeDtypeStruct((NC * NP, 128), jnp.float32),
    mesh=_MESH,
    scratch_types=_AGG_SCRATCH,
)
def _sc_agg1(g_hbm, pack_hbm, zero_hbm, out_hbm, *sc):
    idxs = sc[:_NB]
    buf, acc = sc[_NB], sc[_NB + 1]
    isems = sc[_NB + 2:2 * _NB + 2]
    gsems = sc[2 * _NB + 2:2 * _NB + 4]
    ssems = sc[2 * _NB + 4:2 * _NB + 6]
    c = lax.axis_index("c")
    s = lax.axis_index("s")
    kbase = c * _KCH + s * (_KCH // NS)
    _agg_body(_KCH // NS)(g_hbm, pack_hbm, zero_hbm, out_hbm,
                          idxs, buf, acc, isems, gsems, ssems, kbase, s)
    pltpu.sync_copy(acc.at[pl.ds(s * RPS, RPS)],
                    out_hbm.at[pl.ds(c * NP + s * RPS, RPS)])


@functools.partial(
    pl.kernel,
    out_type=jax.ShapeDtypeStruct((NC * NP, 128), jnp.float32),
    mesh=_MESH,
    scratch_types=_AGG_SCRATCH,
)
def _sc_agg2(g_hbm, pack_hbm, zero_hbm, out_hbm, *sc):
    idxs = sc[:_NB]
    buf, acc = sc[_NB], sc[_NB + 1]
    isems = sc[_NB + 2:2 * _NB + 2]
    gsems = sc[2 * _NB + 2:2 * _NB + 4]
    ssems = sc[2 * _NB + 4:2 * _NB + 6]
    c = lax.axis_index("c")
    s = lax.axis_index("s")
    kbase = c * (_KCH // NC) + s * (_KCH // (NC * NS))
    _agg_body(_KCH // (NC * NS))(g_hbm, pack_hbm, zero_hbm, out_hbm,
                                 idxs, buf, acc, isems, gsems, ssems, kbase, s)
    pltpu.sync_copy(acc.at[pl.ds(s * RPS, RPS)],
                    out_hbm.at[pl.ds(c * NP + s * RPS, RPS)])


_BN = 256
_GRID = (NP // _BN,)


def _p_from_deg(dref):
    deg = dref[0, :, 0] + dref[1, :, 0] + 1.0 + 1e-8
    return lax.rsqrt(deg)


def _tc1_body(x_ref, w_ref, d_ref, h_ref, g_ref):
    p = _p_from_deg(d_ref)
    h = jnp.dot(x_ref[...], w_ref[...], preferred_element_type=jnp.float32)
    h_ref[...] = h
    g = h * p[:, None]
    g_ref[0] = g[:, :128]
    g_ref[1] = g[:, 128:]


_tc1 = pl.pallas_call(
    _tc1_body,
    grid=_GRID,
    in_specs=[
        pl.BlockSpec((_BN, D), lambda i: (i, 0)),
        pl.BlockSpec((D, HID), lambda i: (0, 0)),
        pl.BlockSpec((2, _BN, 16), lambda i: (0, i, 0)),
    ],
    out_specs=[
        pl.BlockSpec((_BN, HID), lambda i: (i, 0)),
        pl.BlockSpec((2, _BN, 128), lambda i: (0, i, 0)),
    ],
    out_shape=[
        jax.ShapeDtypeStruct((NP, HID), jnp.float32),
        jax.ShapeDtypeStruct((2, NP, 128), jnp.float32),
    ],
)


def _tc2_body(a_ref, h1_ref, d_ref, b_ref, w_ref, h2_ref, g2_ref):
    p = _p_from_deg(d_ref)
    agg = jnp.concatenate([a_ref[0], a_ref[1]], axis=1)
    z = p[:, None] * agg + (p * p)[:, None] * h1_ref[...] + b_ref[...]
    z = jnp.maximum(z, 0.0)
    h2 = jnp.dot(z, w_ref[...], preferred_element_type=jnp.float32)
    h2_ref[...] = h2
    g2_ref[:, :C] = h2 * p[:, None]
    g2_ref[:, C:] = jnp.zeros((_BN, 128 - C), jnp.float32)


_tc2 = pl.pallas_call(
    _tc2_body,
    grid=_GRID,
    in_specs=[
        pl.BlockSpec((2, _BN, 128), lambda i: (0, i, 0)),
        pl.BlockSpec((_BN, HID), lambda i: (i, 0)),
        pl.BlockSpec((2, _BN, 16), lambda i: (0, i, 0)),
        pl.BlockSpec((1, HID), lambda i: (0, 0)),
        pl.BlockSpec((HID, C), lambda i: (0, 0)),
    ],
    out_specs=[
        pl.BlockSpec((_BN, C), lambda i: (i, 0)),
        pl.BlockSpec((_BN, 128), lambda i: (i, 0)),
    ],
    out_shape=[
        jax.ShapeDtypeStruct((NP, C), jnp.float32),
        jax.ShapeDtypeStruct((NP, 128), jnp.float32),
    ],
)


def _tc3_body(a_ref, h2_ref, d_ref, b_ref, o_ref):
    p = _p_from_deg(d_ref)
    z = (p[:, None] * (a_ref[0, :, :C] + a_ref[1, :, :C])
         + (p * p)[:, None] * h2_ref[...] + b_ref[...])
    m = jnp.max(z, axis=-1, keepdims=True)
    e = jnp.exp(z - m)
    lse = jnp.log(jnp.sum(e, axis=-1, keepdims=True))
    o_ref[...] = z - m - lse


_tc3 = pl.pallas_call(
    _tc3_body,
    grid=_GRID,
    in_specs=[
        pl.BlockSpec((2, _BN, 128), lambda i: (0, i, 0)),
        pl.BlockSpec((_BN, C), lambda i: (i, 0)),
        pl.BlockSpec((2, _BN, 16), lambda i: (0, i, 0)),
        pl.BlockSpec((1, C), lambda i: (0, 0)),
    ],
    out_specs=pl.BlockSpec((_BN, C), lambda i: (i, 0)),
    out_shape=jax.ShapeDtypeStruct((NP, C), jnp.float32),
)


def kernel(x, edge_index, W1, b1, W2, b2):
    row = edge_index[0].astype(jnp.int32)
    col = edge_index[1].astype(jnp.int32)
    pad = jnp.full((EP - E,), NP - 1, jnp.int32)
    rowp = jnp.concatenate([row, pad])
    colp = jnp.concatenate([col, pad])
    kc = colp.reshape(_KCH, CH)
    kr = rowp.reshape(_KCH, CH)
    pack2 = jnp.stack([kc, kr], axis=1)
    pack1 = jnp.concatenate(
        [pack2, jnp.stack([kc + NP, kr], axis=1)], axis=0)

    xp = jnp.pad(x, ((0, NP - N), (0, 0)))
    ones128 = jnp.ones((CH, 128), jnp.float32)
    z1 = jnp.zeros((RPS, 128), jnp.float32)

    degp = _sc_degree(rowp, ones128, z1).reshape(2, NP, 128)[:, :, :16]
    h1, g1 = _tc1(xp, W1, degp)
    acc1 = _sc_agg1(g1.reshape(2 * NP, 128), pack1, z1).reshape(2, NP, 128)
    h2, g2 = _tc2(acc1, h1, degp, b1.reshape(1, HID), W2)
    acc2 = _sc_agg2(g2, pack2, z1).reshape(2, NP, 128)
    out = _tc3(acc2, h2, degp, b2.reshape(1, C))
    return out[:N]
